# baseline (speedup 1.0000x reference)
import functools

import jax
import jax.numpy as jnp
from jax import lax
from jax.experimental import pallas as pl
from jax.experimental.pallas import tpu as pltpu
from jax.experimental.pallas import tpu_sc as plsc

_IDX_CHUNK = 128
_HALF = 256


def _sc_gather3(W128, V128, hi_u, hi_b, hi_w, bpw, n_chunks):
    B = hi_u.shape[0] * _IDX_CHUNK
    mesh = plsc.VectorSubcoreMesh(core_axis_name="c", subcore_axis_name="s")
    info = plsc.get_sparse_core_info()
    NC = info.num_cores

    out_t = (jax.ShapeDtypeStruct((B, 128), jnp.float32),) * 3
    scratch_t = [
        pltpu.VMEM((n_chunks, _IDX_CHUNK), jnp.int32),
        pltpu.VMEM((n_chunks, _IDX_CHUNK), jnp.int32),
        pltpu.VMEM((n_chunks, _IDX_CHUNK), jnp.int32),
        pltpu.VMEM((_HALF, 128), jnp.float32),
        pltpu.VMEM((_HALF, 128), jnp.float32),
        pltpu.VMEM((_HALF, 128), jnp.float32),
        pltpu.SemaphoreType.DMA,
        pltpu.SemaphoreType.DMA,
        pltpu.SemaphoreType.DMA,
        pltpu.SemaphoreType.DMA,
        pltpu.SemaphoreType.DMA,
        pltpu.SemaphoreType.DMA,
    ]
    n_half = bpw // _HALF
    chunks_per_half = _HALF // _IDX_CHUNK

    @functools.partial(pl.kernel, mesh=mesh, out_type=out_t, scratch_types=scratch_t)
    def k(w_hbm, v_hbm, iu_hbm, ib_hbm, iw_hbm, ou_hbm, ob_hbm, ow_hbm,
          iu_v, ib_v, iw_v, r0, r1, r2, g0, g1, g2, w0, w1, w2):
        wid = lax.axis_index("s") * NC + lax.axis_index("c")
        ibase = wid * n_chunks
        rbase = wid * bpw
        pltpu.sync_copy(iu_hbm.at[pl.ds(ibase, n_chunks)], iu_v)
        pltpu.sync_copy(ib_hbm.at[pl.ds(ibase, n_chunks)], ib_v)
        pltpu.sync_copy(iw_hbm.at[pl.ds(ibase, n_chunks)], iw_v)
        rings = (r0, r1, r2)
        gsems = (g0, g1, g2)
        wsems = (w0, w1, w2)
        passes = [(t, h) for t in range(3) for h in range(n_half)]
        tables = (w_hbm, v_hbm, v_hbm)
        idxs = (iu_v, ib_v, iw_v)
        outs = (ou_hbm, ob_hbm, ow_hbm)
        for p, (t, h) in enumerate(passes):
            s = p % 3
            buf, gsem, wsem = rings[s], gsems[s], wsems[s]
            if p >= 3:
                pt, ph = passes[p - 3]
                pltpu.make_async_copy(
                    buf, outs[pt].at[pl.ds(rbase + ph * _HALF, _HALF)], wsem
                ).wait()
            for c in range(chunks_per_half):
                j = h * chunks_per_half + c
                dst = pl.ds(c * _IDX_CHUNK, _IDX_CHUNK)
                pltpu.async_copy(tables[t].at[idxs[t].at[j]], buf.at[dst], gsem)
            for c in range(chunks_per_half):
                dst = pl.ds(c * _IDX_CHUNK, _IDX_CHUNK)
                pltpu.make_async_copy(
                    tables[t].at[idxs[t].at[h * chunks_per_half + c]],
                    buf.at[dst], gsem).wait()
            pltpu.async_copy(
                buf, outs[t].at[pl.ds(rbase + h * _HALF, _HALF)], wsem)
        for p in range(len(passes) - 3, len(passes)):
            t, h = passes[p]
            s = p % 3
            pltpu.make_async_copy(
                rings[s], outs[t].at[pl.ds(rbase + h * _HALF, _HALF)], wsems[s]
            ).wait()

    return k(W128, V128, hi_u, hi_b, hi_w)


def _tc_outer(wu128, vb128, vw128, iu, ib, iw, BB, K):
    B = wu128.shape[0]
    KK = K * K
    nsel = 128 // K

    def body(wu_ref, vb_ref, vw_ref, iu_ref, ib_ref, iw_ref, fb_ref, fw_ref):
        def select(rows_ref, idx_ref):
            lo = lax.broadcast_in_dim(idx_ref[...] % nsel, (BB, K), (0, 1))
            acc = jnp.zeros((BB, K), jnp.float32)
            for c in range(nsel):
                acc += jnp.where(lo == c, rows_ref[:, c * K:(c + 1) * K], 0.0)
            return acc

        wu = select(wu_ref, iu_ref)
        vb = select(vb_ref, ib_ref)
        vw = select(vw_ref, iw_ref)
        col = lax.broadcasted_iota(jnp.int32, (K, KK), 1)
        row = lax.broadcasted_iota(jnp.int32, (K, KK), 0)
        rep_m = (col // K == row).astype(jnp.float32)
        til_m = (col % K == row).astype(jnp.float32)
        rep = jnp.dot(wu, rep_m, preferred_element_type=jnp.float32)
        fb_ref[...] = rep * jnp.dot(vb, til_m, preferred_element_type=jnp.float32)
        fw_ref[...] = rep * jnp.dot(vw, til_m, preferred_element_type=jnp.float32)

    row_spec = pl.BlockSpec((BB, 128), lambda i: (i, 0))
    idx_spec = pl.BlockSpec((BB, 1), lambda i: (i, 0))
    out_spec = pl.BlockSpec((BB, KK), lambda i: (i, 0))
    return pl.pallas_call(
        body,
        grid=(B // BB,),
        in_specs=[row_spec] * 3 + [idx_spec] * 3,
        out_specs=[out_spec, out_spec],
        out_shape=[jax.ShapeDtypeStruct((B, KK), jnp.float32)] * 2,
    )(wu128, vb128, vw128, iu, ib, iw)


def kernel(input_user, best_movie, worst_movie, W, V):
    B = input_user.shape[0]
    K = W.shape[1]
    nsel = 128 // K
    iu = input_user.reshape(B, 1).astype(jnp.int32)
    ib = best_movie.reshape(B, 1).astype(jnp.int32)
    iw = worst_movie.reshape(B, 1).astype(jnp.int32)
    hi_u = (iu // nsel).reshape(B // _IDX_CHUNK, _IDX_CHUNK)
    hi_b = (ib // nsel).reshape(B // _IDX_CHUNK, _IDX_CHUNK)
    hi_w = (iw // nsel).reshape(B // _IDX_CHUNK, _IDX_CHUNK)
    W128 = W.reshape(-1, 128)
    V128 = V.reshape(-1, 128)

    info = plsc.get_sparse_core_info()
    nw = info.num_cores * info.num_subcores
    bpw = B // nw
    n_chunks = bpw // _IDX_CHUNK

    wu128, vb128, vw128 = _sc_gather3(W128, V128, hi_u, hi_b, hi_w, bpw, n_chunks)
    fb, fw = _tc_outer(wu128, vb128, vw128, iu, ib, iw, BB=512, K=K)
    return fb.reshape(B, K, K), fw.reshape(B, K, K)


# speedup vs baseline: 1.0793x; 1.0793x over previous
import functools

import jax
import jax.numpy as jnp
from jax import lax
from jax.experimental import pallas as pl
from jax.experimental.pallas import tpu as pltpu
from jax.experimental.pallas import tpu_sc as plsc

_IDX_CHUNK = 128
_PACK = 4
_NBUF = 4


def _sc_gather3(W4, V4, iu, ib, iw, bpw, n_chunks):
    D = W4.shape[1]
    B = iu.shape[0] * _IDX_CHUNK
    mesh = plsc.VectorSubcoreMesh(core_axis_name="c", subcore_axis_name="s")
    info = plsc.get_sparse_core_info()
    NC = info.num_cores

    out_t = (jax.ShapeDtypeStruct((B, D), jnp.float32),) * 3
    scratch_t = (
        [pltpu.VMEM((n_chunks, _IDX_CHUNK), jnp.int32)] * 3
        + [pltpu.VMEM((_IDX_CHUNK, D), jnp.float32)] * _NBUF
        + [pltpu.SemaphoreType.DMA] * _NBUF
        + [pltpu.SemaphoreType.DMA] * _NBUF
    )

    @functools.partial(pl.kernel, mesh=mesh, out_type=out_t,
                       scratch_types=scratch_t)
    def k(w_hbm, v_hbm, iu_hbm, ib_hbm, iw_hbm, ou_hbm, ob_hbm, ow_hbm,
          iu_v, ib_v, iw_v, *rs):
        bufs = rs[:_NBUF]
        gsems = rs[_NBUF:2 * _NBUF]
        wsems = rs[2 * _NBUF:]
        wid = lax.axis_index("s") * NC + lax.axis_index("c")
        ibase = wid * n_chunks
        base = wid * bpw
        pltpu.sync_copy(iu_hbm.at[pl.ds(ibase, n_chunks)], iu_v)
        pltpu.sync_copy(ib_hbm.at[pl.ds(ibase, n_chunks)], ib_v)
        pltpu.sync_copy(iw_hbm.at[pl.ds(ibase, n_chunks)], iw_v)
        tabs = (w_hbm, v_hbm, v_hbm)
        idxs = (iu_v, ib_v, iw_v)
        outs = (ou_hbm, ob_hbm, ow_hbm)
        q = 0
        pending = [None] * _NBUF
        for t in range(3):
            for c in range(n_chunks):
                b = q % _NBUF
                if pending[b] is not None:
                    pltpu.make_async_copy(bufs[b], pending[b], wsems[b]).wait()
                pltpu.async_copy(tabs[t].at[idxs[t].at[c]], bufs[b], gsems[b])
                pltpu.make_async_copy(
                    tabs[t].at[idxs[t].at[c]], bufs[b], gsems[b]).wait()
                dst = outs[t].at[pl.ds(base + c * _IDX_CHUNK, _IDX_CHUNK)]
                pltpu.async_copy(bufs[b], dst, wsems[b])
                pending[b] = dst
                q += 1
        for b in range(_NBUF):
            if pending[b] is not None:
                pltpu.make_async_copy(bufs[b], pending[b], wsems[b]).wait()

    return k(W4, V4, iu, ib, iw)


def _tc_outer_t(gu, gb, gw, mu, mb, mw, K, BB):
    B, D = gu.shape
    KK = K * K

    def body(gu_ref, gb_ref, gw_ref, mu_ref, mb_ref, mw_ref, fb_ref, fw_ref):
        def pick(g_ref, m_ref):
            m = m_ref[...]
            acc = jnp.where(m == 0, g_ref[:, 0:K], 0.0)
            for s in range(1, _PACK):
                acc += jnp.where(m == s, g_ref[:, s * K:(s + 1) * K], 0.0)
            return acc

        row = lax.broadcasted_iota(jnp.int32, (KK, K), 0)
        col = lax.broadcasted_iota(jnp.int32, (KK, K), 1)
        rep_m = (row // K == col).astype(jnp.float32)
        til_m = (row % K == col).astype(jnp.float32)
        dn = (((1,), (1,)), ((), ()))
        rep = lax.dot_general(rep_m, pick(gu_ref, mu_ref), dn,
                              preferred_element_type=jnp.float32)
        fb_ref[...] = rep * lax.dot_general(til_m, pick(gb_ref, mb_ref), dn,
                                            preferred_element_type=jnp.float32)
        fw_ref[...] = rep * lax.dot_general(til_m, pick(gw_ref, mw_ref), dn,
                                            preferred_element_type=jnp.float32)

    g_spec = pl.BlockSpec((BB, D), lambda i: (i, 0))
    m_spec = pl.BlockSpec((BB, 1), lambda i: (i, 0))
    out_spec = pl.BlockSpec((KK, BB), lambda i: (0, i))
    return pl.pallas_call(
        body,
        grid=(B // BB,),
        in_specs=[g_spec] * 3 + [m_spec] * 3,
        out_specs=[out_spec, out_spec],
        out_shape=[jax.ShapeDtypeStruct((KK, B), jnp.float32)] * 2,
    )(gu, gb, gw, mu, mb, mw)


def kernel(input_user, best_movie, worst_movie, W, V):
    B = input_user.shape[0]
    N, K = W.shape
    iu = input_user.reshape(B).astype(jnp.int32)
    ib = best_movie.reshape(B).astype(jnp.int32)
    iw = worst_movie.reshape(B).astype(jnp.int32)

    W4 = W.reshape(N // _PACK, _PACK * K)
    V4 = V.reshape(N // _PACK, _PACK * K)
    rows = lambda i: (i // _PACK).reshape(B // _IDX_CHUNK, _IDX_CHUNK)
    mods = lambda i: (i % _PACK).reshape(B, 1)

    info = plsc.get_sparse_core_info()
    nw = info.num_cores * info.num_subcores
    bpw = B // nw
    n_chunks = bpw // _IDX_CHUNK

    gu, gb, gw = _sc_gather3(W4, V4, rows(iu), rows(ib), rows(iw),
                             bpw, n_chunks)
    fbT, fwT = _tc_outer_t(gu, gb, gw, mods(iu), mods(ib), mods(iw),
                           K, BB=512)
    fb = fbT.reshape(K, K, B).transpose(2, 0, 1)
    fw = fwT.reshape(K, K, B).transpose(2, 0, 1)
    return fb, fw


# speedup vs baseline: 1.1214x; 1.0390x over previous
import functools

import jax
import jax.numpy as jnp
from jax import lax
from jax.experimental import pallas as pl
from jax.experimental.pallas import tpu as pltpu
from jax.experimental.pallas import tpu_sc as plsc

_IDX_CHUNK = 128
_PACK = 4
_NBUF = 4


def _sc_gather3(W4, V4, iu, ib, iw, bpw, n_chunks):
    D = W4.shape[1]
    B = iu.shape[0] * _IDX_CHUNK
    mesh = plsc.VectorSubcoreMesh(core_axis_name="c", subcore_axis_name="s")
    info = plsc.get_sparse_core_info()
    NC = info.num_cores

    out_t = (jax.ShapeDtypeStruct((B, D), jnp.float32),) * 3
    scratch_t = (
        [pltpu.VMEM((n_chunks, _IDX_CHUNK), jnp.int32)] * 3
        + [pltpu.VMEM((_IDX_CHUNK, D), jnp.float32)] * _NBUF
        + [pltpu.SemaphoreType.DMA] * _NBUF
        + [pltpu.SemaphoreType.DMA] * _NBUF
    )

    @functools.partial(pl.kernel, mesh=mesh, out_type=out_t,
                       scratch_types=scratch_t)
    def k(w_hbm, v_hbm, iu_hbm, ib_hbm, iw_hbm, ou_hbm, ob_hbm, ow_hbm,
          iu_v, ib_v, iw_v, *rs):
        bufs = rs[:_NBUF]
        gsems = rs[_NBUF:2 * _NBUF]
        wsems = rs[2 * _NBUF:]
        wid = lax.axis_index("s") * NC + lax.axis_index("c")
        ibase = wid * n_chunks
        base = wid * bpw
        pltpu.sync_copy(iu_hbm.at[pl.ds(ibase, n_chunks)], iu_v)
        pltpu.sync_copy(ib_hbm.at[pl.ds(ibase, n_chunks)], ib_v)
        pltpu.sync_copy(iw_hbm.at[pl.ds(ibase, n_chunks)], iw_v)
        tabs = (w_hbm, v_hbm, v_hbm)
        idxs = (iu_v, ib_v, iw_v)
        outs = (ou_hbm, ob_hbm, ow_hbm)
        q = 0
        pending = [None] * _NBUF
        for t in range(3):
            for c in range(n_chunks):
                b = q % _NBUF
                if pending[b] is not None:
                    pltpu.make_async_copy(bufs[b], pending[b], wsems[b]).wait()
                pltpu.async_copy(tabs[t].at[idxs[t].at[c]], bufs[b], gsems[b])
                pltpu.make_async_copy(
                    tabs[t].at[idxs[t].at[c]], bufs[b], gsems[b]).wait()
                dst = outs[t].at[pl.ds(base + c * _IDX_CHUNK, _IDX_CHUNK)]
                pltpu.async_copy(bufs[b], dst, wsems[b])
                pending[b] = dst
                q += 1
        for b in range(_NBUF):
            if pending[b] is not None:
                pltpu.make_async_copy(bufs[b], pending[b], wsems[b]).wait()

    return k(W4, V4, iu, ib, iw)


def _tc_pack4(WT, RB):
    K, N = WT.shape
    grid = (N + _PACK * RB - 1) // (_PACK * RB)

    def body(in_ref, out_ref):
        xt = jnp.transpose(in_ref[...])
        out_ref[...] = jnp.concatenate(
            [xt[a * RB:(a + 1) * RB, :] for a in range(_PACK)], axis=1)

    return pl.pallas_call(
        body,
        grid=(grid,),
        in_specs=[pl.BlockSpec((K, _PACK * RB), lambda i: (0, i))],
        out_specs=pl.BlockSpec((RB, _PACK * K), lambda i: (i, 0)),
        out_shape=jax.ShapeDtypeStruct((grid * RB, _PACK * K), jnp.float32),
    )(WT)


def _tc_outer_t(gu, gb, gw, mu, mb, mw, K, BB):
    B, D = gu.shape
    KK = K * K

    def body(gu_ref, gb_ref, gw_ref, mu_ref, mb_ref, mw_ref, fb_ref, fw_ref):
        def pick(g_ref, m_ref):
            m = m_ref[...]
            acc = jnp.where(m == 0, g_ref[:, 0:K], 0.0)
            for s in range(1, _PACK):
                acc += jnp.where(m == s, g_ref[:, s * K:(s + 1) * K], 0.0)
            return acc

        row = lax.broadcasted_iota(jnp.int32, (KK, K), 0)
        col = lax.broadcasted_iota(jnp.int32, (KK, K), 1)
        rep_m = (row // K == col).astype(jnp.float32)
        til_m = (row % K == col).astype(jnp.float32)
        dn = (((1,), (1,)), ((), ()))
        rep = lax.dot_general(rep_m, pick(gu_ref, mu_ref), dn,
                              preferred_element_type=jnp.float32)
        fb_ref[...] = rep * lax.dot_general(til_m, pick(gb_ref, mb_ref), dn,
                                            preferred_element_type=jnp.float32)
        fw_ref[...] = rep * lax.dot_general(til_m, pick(gw_ref, mw_ref), dn,
                                            preferred_element_type=jnp.float32)

    g_spec = pl.BlockSpec((BB, D), lambda i: (i, 0))
    m_spec = pl.BlockSpec((BB, 1), lambda i: (i, 0))
    out_spec = pl.BlockSpec((KK, BB), lambda i: (0, i))
    return pl.pallas_call(
        body,
        grid=(B // BB,),
        in_specs=[g_spec] * 3 + [m_spec] * 3,
        out_specs=[out_spec, out_spec],
        out_shape=[jax.ShapeDtypeStruct((KK, B), jnp.float32)] * 2,
    )(gu, gb, gw, mu, mb, mw)


def kernel(input_user, best_movie, worst_movie, W, V):
    B = input_user.shape[0]
    N, K = W.shape
    iu = input_user.reshape(B).astype(jnp.int32)
    ib = best_movie.reshape(B).astype(jnp.int32)
    iw = worst_movie.reshape(B).astype(jnp.int32)

    RB = 512
    W4 = _tc_pack4(W.T, RB=RB)
    V4 = _tc_pack4(V.T, RB=RB)
    rows = lambda i: ((i // (_PACK * RB)) * RB
                      + i % RB).reshape(B // _IDX_CHUNK, _IDX_CHUNK)
    mods = lambda i: ((i // RB) % _PACK).reshape(B, 1)

    info = plsc.get_sparse_core_info()
    nw = info.num_cores * info.num_subcores
    bpw = B // nw
    n_chunks = bpw // _IDX_CHUNK

    gu, gb, gw = _sc_gather3(W4, V4, rows(iu), rows(ib), rows(iw),
                             bpw, n_chunks)
    fbT, fwT = _tc_outer_t(gu, gb, gw, mods(iu), mods(ib), mods(iw),
                           K, BB=512)
    fb = fbT.reshape(K, K, B).transpose(2, 0, 1)
    fw = fwT.reshape(K, K, B).transpose(2, 0, 1)
    return fb, fw
